# baseline (speedup 1.0000x reference)
import functools

import jax
import jax.numpy as jnp
from jax import lax
from jax.experimental import pallas as pl
from jax.experimental.pallas import tpu as pltpu
from jax.experimental.pallas import tpu_sc as plsc

_INFO = plsc.get_sparse_core_info()
_NC = _INFO.num_cores

_CHUNK_ROWS = 256
_NBUF = 3


def _sc_copy(table, seq_len):
    d_model = table.shape[1]
    rows_per_w = seq_len // _NC
    n_chunks = rows_per_w // _CHUNK_ROWS
    nbuf = min(_NBUF, n_chunks)

    mesh = plsc.ScalarSubcoreMesh(axis_name="c")

    @functools.partial(
        pl.kernel,
        mesh=mesh,
        out_type=jax.ShapeDtypeStruct((seq_len, d_model), table.dtype),
        scratch_types=(
            [pltpu.VMEM_SHARED((_CHUNK_ROWS, d_model), table.dtype)] * nbuf
            + [pltpu.SemaphoreType.DMA] * (2 * nbuf)
        ),
    )
    def body(table_hbm, out_hbm, *scratch):
        bufs = scratch[:nbuf]
        lsem = scratch[nbuf : 2 * nbuf]
        ssem = scratch[2 * nbuf :]
        base = lax.axis_index("c") * rows_per_w

        def load(c):
            return pltpu.make_async_copy(
                table_hbm.at[pl.ds(base + c * _CHUNK_ROWS, _CHUNK_ROWS)],
                bufs[c % nbuf],
                lsem[c % nbuf],
            )

        def store(c):
            return pltpu.make_async_copy(
                bufs[c % nbuf],
                out_hbm.at[pl.ds(base + c * _CHUNK_ROWS, _CHUNK_ROWS)],
                ssem[c % nbuf],
            )

        for c in range(nbuf):
            load(c).start()
        for c in range(n_chunks):
            load(c).wait()
            store(c).start()
            if c + nbuf < n_chunks:
                store(c).wait()
                load(c + nbuf).start()
        for c in range(max(0, n_chunks - nbuf), n_chunks):
            store(c).wait()

    return body(table)


def kernel(x, table):
    seq_len = x.shape[1]
    out = _sc_copy(table, seq_len)
    return out[None]


# speedup vs baseline: 1.0770x; 1.0770x over previous
import functools

import jax
import jax.numpy as jnp
from jax import lax
from jax.experimental import pallas as pl
from jax.experimental.pallas import tpu as pltpu
from jax.experimental.pallas import tpu_sc as plsc

_INFO = plsc.get_sparse_core_info()
_NC = _INFO.num_cores
_NS = _INFO.num_subcores
_NW = _NC * _NS

_CHUNK_ROWS = 32
_NBUF = 3


def _sc_copy(table, seq_len):
    d_model = table.shape[1]
    rows_per_w = seq_len // _NW
    n_chunks = rows_per_w // _CHUNK_ROWS

    mesh = plsc.VectorSubcoreMesh(core_axis_name="c", subcore_axis_name="s")
    nbuf = min(_NBUF, n_chunks)

    @functools.partial(
        pl.kernel,
        mesh=mesh,
        out_type=jax.ShapeDtypeStruct((seq_len, d_model), table.dtype),
        scratch_types=(
            [pltpu.VMEM((_CHUNK_ROWS, d_model), table.dtype)] * nbuf
            + [pltpu.SemaphoreType.DMA] * (2 * nbuf)
        ),
    )
    def body(table_hbm, out_hbm, *scratch):
        bufs = scratch[:nbuf]
        lsem = scratch[nbuf : 2 * nbuf]
        ssem = scratch[2 * nbuf :]
        wid = lax.axis_index("s") * _NC + lax.axis_index("c")
        base = wid * rows_per_w

        def load(c):
            return pltpu.make_async_copy(
                table_hbm.at[pl.ds(base + c * _CHUNK_ROWS, _CHUNK_ROWS)],
                bufs[c % nbuf],
                lsem[c % nbuf],
            )

        def store(c):
            return pltpu.make_async_copy(
                bufs[c % nbuf],
                out_hbm.at[pl.ds(base + c * _CHUNK_ROWS, _CHUNK_ROWS)],
                ssem[c % nbuf],
            )

        for c in range(nbuf):
            load(c).start()
        for c in range(n_chunks):
            load(c).wait()
            store(c).start()
            if c + nbuf < n_chunks:
                store(c).wait()
                load(c + nbuf).start()
        for c in range(max(0, n_chunks - nbuf), n_chunks):
            store(c).wait()

    return body(table)


def kernel(x, table):
    seq_len = x.shape[1]
    out = _sc_copy(table, seq_len)
    return out[None]
